# baseline (speedup 1.0000x reference)
import functools

import jax
import jax.numpy as jnp
from jax import lax
from jax.experimental import pallas as pl
from jax.experimental.pallas import tpu as pltpu
from jax.experimental.pallas import tpu_sc as plsc

_NC, _NS = 2, 16
_NW = _NC * _NS


def _film_tables_kernel(mt_ref, ws_ref, wh_ref, bs_ref, bh_ref,
                        scale_ref, shift_ref):
    mt = mt_ref[...]
    dn = (((1,), (1,)), ((), ()))
    scale_ref[...] = lax.dot_general(
        mt, ws_ref[...], dn, preferred_element_type=jnp.float32) + bs_ref[...]
    shift_ref[...] = lax.dot_general(
        mt, wh_ref[...], dn, preferred_element_type=jnp.float32) + bh_ref[...]


def _make_sc_row_gather(B, D, dtype, chunk, with_aux):
    rows_per_w = B // _NW
    nchunks = rows_per_w // chunk
    mesh = plsc.VectorSubcoreMesh(core_axis_name="c", subcore_axis_name="s")
    out_type = [jax.ShapeDtypeStruct((B, D), dtype)]
    scratch = [pltpu.VMEM((chunk,), jnp.int32),
               pltpu.VMEM((chunk, D), dtype),
               pltpu.SemaphoreType.DMA]
    if with_aux:
        out_type.append(jax.ShapeDtypeStruct((B,), jnp.int32))
        scratch.append(pltpu.VMEM((chunk,), jnp.int32))

    @functools.partial(pl.kernel, mesh=mesh, out_type=tuple(out_type),
                       scratch_types=scratch)
    def gather(*refs):
        if with_aux:
            table_hbm, idx_hbm, aux_hbm, out_hbm, aux_out_hbm, \
                idx_v, rows_v, sem, aux_v = refs
        else:
            table_hbm, idx_hbm, out_hbm, idx_v, rows_v, sem = refs
        wid = lax.axis_index("s") * _NC + lax.axis_index("c")
        base = wid * rows_per_w
        for c in range(nchunks):
            off = base + c * chunk
            pltpu.sync_copy(idx_hbm.at[pl.ds(off, chunk)], idx_v)
            pltpu.async_copy(table_hbm.at[idx_v], rows_v, sem).wait()
            pltpu.sync_copy(rows_v, out_hbm.at[pl.ds(off, chunk)])
            if with_aux:
                pltpu.async_copy(aux_hbm.at[idx_v], aux_v, sem).wait()
                pltpu.sync_copy(aux_v, aux_out_hbm.at[pl.ds(off, chunk)])

    return gather


def _grouped_kernel(s_ref, x_ref, mode_ref, w0_ref, w1_ref, w2_ref,
                    b0_ref, b1_ref, b2_ref, st_ref, ht_ref, out_ref):
    i = pl.program_id(0)
    BM = x_ref.shape[0]
    d1 = w1_ref.shape[1]
    d2 = w2_ref.shape[1]
    n0 = s_ref[0]
    n01 = s_ref[1]
    lo = i * BM
    row = lo + lax.broadcasted_iota(jnp.int32, (BM, 1), 0)
    x = x_ref[...]
    dn = (((1,), (1,)), ((), ()))

    out_ref[...] = jnp.zeros_like(out_ref)

    @pl.when(n0 > lo)
    def _():
        m = (row < n0).astype(jnp.float32)
        p = lax.dot_general(x, w0_ref[...], dn,
                            preferred_element_type=jnp.float32)
        out_ref[...] += m * (p + b0_ref[...])

    @pl.when((n01 > lo) & (n0 < lo + BM))
    def _():
        m = ((row >= n0) & (row < n01)).astype(jnp.float32)
        p = lax.dot_general(x[:, :d1], w1_ref[...], dn,
                            preferred_element_type=jnp.float32)
        out_ref[...] += m * (p + b1_ref[...])

    @pl.when(n01 < lo + BM)
    def _():
        m = (row >= n01).astype(jnp.float32)
        p = lax.dot_general(x[:, :d2], w2_ref[...], dn,
                            preferred_element_type=jnp.float32)
        out_ref[...] += m * (p + b2_ref[...])

    modes = mode_ref[0, 0, :]
    n_modes = st_ref.shape[0]
    oh = (modes[:, None] == lax.broadcasted_iota(
        jnp.int32, (BM, n_modes), 1)).astype(jnp.float32)
    scale = jnp.dot(oh, st_ref[...], preferred_element_type=jnp.float32)
    shift = jnp.dot(oh, ht_ref[...], preferred_element_type=jnp.float32)
    out_ref[...] = out_ref[...] * (1.0 + scale) + shift


def kernel(features, modality_ids, mode_ids, W0, b0, W1, b1, W2, b2,
           mode_table, Ws, bs, Wh, bh):
    B, D = features.shape
    L = W0.shape[0]
    n_modes = mode_table.shape[0]

    scale_t, shift_t = pl.pallas_call(
        _film_tables_kernel,
        out_shape=(jax.ShapeDtypeStruct((n_modes, L), jnp.float32),
                   jax.ShapeDtypeStruct((n_modes, L), jnp.float32)),
    )(mode_table, Ws, Wh, bs.reshape(1, L), bh.reshape(1, L))

    is0 = modality_ids == 0
    is1 = modality_ids == 1
    n0 = jnp.sum(is0).astype(jnp.int32)
    n01 = jnp.sum(modality_ids < 2).astype(jnp.int32)
    c0 = jnp.cumsum(is0.astype(jnp.int32))
    c1 = jnp.cumsum(is1.astype(jnp.int32))
    c2 = jnp.cumsum((modality_ids == 2).astype(jnp.int32))
    pos = jnp.where(is0, c0 - 1,
                    jnp.where(is1, n0 + c1 - 1, n01 + c2 - 1)).astype(jnp.int32)
    iota = lax.iota(jnp.int32, B)
    order = jnp.zeros((B,), jnp.int32).at[pos].set(iota)
    sizes = jnp.stack([n0, n01])

    xi = lax.bitcast_convert_type(
        features.astype(jnp.bfloat16).reshape(B, D // 2, 2), jnp.int32)
    xs_i, mode_s = _make_sc_row_gather(B, D // 2, jnp.int32, 64, True)(
        xi, order, mode_ids)
    xs = lax.bitcast_convert_type(xs_i, jnp.bfloat16).reshape(B, D)

    BM = 256
    NM = B // BM
    grid_spec = pltpu.PrefetchScalarGridSpec(
        num_scalar_prefetch=1,
        grid=(NM,),
        in_specs=[
            pl.BlockSpec((BM, D), lambda i, s: (i, 0)),
            pl.BlockSpec((1, 1, BM), lambda i, s: (i, 0, 0)),
            pl.BlockSpec((L, D), lambda i, s: (0, 0)),
            pl.BlockSpec((L, W1.shape[1]), lambda i, s: (0, 0)),
            pl.BlockSpec((L, W2.shape[1]), lambda i, s: (0, 0)),
            pl.BlockSpec((1, L), lambda i, s: (0, 0)),
            pl.BlockSpec((1, L), lambda i, s: (0, 0)),
            pl.BlockSpec((1, L), lambda i, s: (0, 0)),
            pl.BlockSpec((n_modes, L), lambda i, s: (0, 0)),
            pl.BlockSpec((n_modes, L), lambda i, s: (0, 0)),
        ],
        out_specs=pl.BlockSpec((BM, L), lambda i, s: (i, 0)),
    )
    ys = pl.pallas_call(
        _grouped_kernel,
        grid_spec=grid_spec,
        out_shape=jax.ShapeDtypeStruct((B, L), jnp.float32),
    )(sizes, xs, mode_s.reshape(NM, 1, BM),
      W0.astype(jnp.bfloat16), W1.astype(jnp.bfloat16),
      W2.astype(jnp.bfloat16),
      b0.reshape(1, L), b1.reshape(1, L), b2.reshape(1, L), scale_t, shift_t)

    out = _make_sc_row_gather(B, L, jnp.float32, 32, False)(ys, pos)
    return out[0] if isinstance(out, (tuple, list)) else out


# speedup vs baseline: 4.1551x; 4.1551x over previous
import jax
import jax.numpy as jnp
from jax import lax
from jax.experimental import pallas as pl


def _tables_kernel(mt_ref, ws_ref, wh_ref, bs_ref, bh_ref, scale_ref, shift_ref):
    mt = mt_ref[...]
    dn = (((1,), (1,)), ((), ()))
    scale = lax.dot_general(
        mt, ws_ref[...], dn, preferred_element_type=jnp.float32) + bs_ref[...]
    shift = lax.dot_general(
        mt, wh_ref[...], dn, preferred_element_type=jnp.float32) + bh_ref[...]
    scale_ref[...] = scale.astype(jnp.bfloat16)
    shift_ref[...] = shift.astype(jnp.bfloat16)


def _main_kernel(x_ref, mod_ref, mode_ref, w0_ref, w1_ref, w2_ref,
                 b0_ref, b1_ref, b2_ref, scale_t_ref, shift_t_ref, out_ref):
    x = x_ref[...].astype(jnp.bfloat16)
    d1 = w1_ref.shape[1]
    d2 = w2_ref.shape[1]
    dn = (((1,), (1,)), ((), ()))
    p0 = lax.dot_general(x, w0_ref[...], dn, preferred_element_type=jnp.float32)
    p1 = lax.dot_general(x[:, :d1], w1_ref[...], dn,
                         preferred_element_type=jnp.float32)
    p2 = lax.dot_general(x[:, :d2], w2_ref[...], dn,
                         preferred_element_type=jnp.float32)

    mids = mod_ref[0, 0, :][:, None]
    content = jnp.where(mids == 0, p0 + b0_ref[...],
                        jnp.where(mids == 1, p1 + b1_ref[...],
                                  p2 + b2_ref[...]))

    modes = mode_ref[0, 0, :]
    n_modes = scale_t_ref.shape[0]
    oh = (modes[:, None] == lax.broadcasted_iota(
        jnp.int32, (modes.shape[0], n_modes), 1)).astype(jnp.bfloat16)
    scale = lax.dot_general(oh, scale_t_ref[...], (((1,), (0,)), ((), ())),
                            preferred_element_type=jnp.float32)
    shift = lax.dot_general(oh, shift_t_ref[...], (((1,), (0,)), ((), ())),
                            preferred_element_type=jnp.float32)

    out_ref[...] = content * (1.0 + scale) + shift


def kernel(features, modality_ids, mode_ids, W0, b0, W1, b1, W2, b2,
           mode_table, Ws, bs, Wh, bh):
    B, D = features.shape
    L = W0.shape[0]
    n_modes = mode_table.shape[0]

    scale_t, shift_t = pl.pallas_call(
        _tables_kernel,
        out_shape=(jax.ShapeDtypeStruct((n_modes, L), jnp.bfloat16),
                   jax.ShapeDtypeStruct((n_modes, L), jnp.bfloat16)),
    )(mode_table, Ws, Wh, bs.reshape(1, L), bh.reshape(1, L))

    BM = 512
    NM = B // BM
    w0b = W0.astype(jnp.bfloat16)
    w1b = W1.astype(jnp.bfloat16)
    w2b = W2.astype(jnp.bfloat16)
    mod3 = modality_ids.reshape(NM, 1, BM)
    mode3 = mode_ids.reshape(NM, 1, BM)

    out = pl.pallas_call(
        _main_kernel,
        grid=(NM,),
        in_specs=[
            pl.BlockSpec((BM, D), lambda i: (i, 0)),
            pl.BlockSpec((1, 1, BM), lambda i: (i, 0, 0)),
            pl.BlockSpec((1, 1, BM), lambda i: (i, 0, 0)),
            pl.BlockSpec((L, D), lambda i: (0, 0)),
            pl.BlockSpec((L, W1.shape[1]), lambda i: (0, 0)),
            pl.BlockSpec((L, W2.shape[1]), lambda i: (0, 0)),
            pl.BlockSpec((1, L), lambda i: (0, 0)),
            pl.BlockSpec((1, L), lambda i: (0, 0)),
            pl.BlockSpec((1, L), lambda i: (0, 0)),
            pl.BlockSpec((n_modes, L), lambda i: (0, 0)),
            pl.BlockSpec((n_modes, L), lambda i: (0, 0)),
        ],
        out_specs=pl.BlockSpec((BM, L), lambda i: (i, 0)),
        out_shape=jax.ShapeDtypeStruct((B, L), jnp.float32),
    )(features, mod3, mode3, w0b, w1b, w2b,
      b0.reshape(1, L), b1.reshape(1, L), b2.reshape(1, L), scale_t, shift_t)
    return out
